# Initial kernel scaffold; baseline (speedup 1.0000x reference)
import jax
import jax.numpy as jnp
from jax.experimental import pallas as pl


def kernel(onehot_ids, v_tables, b_tables, W1, b1, W2, b2, W3, b3, global_bias):
    raise NotImplementedError("write your pallas kernel here")


# speedup vs baseline: 1.1463x; 1.1463x over previous
import functools

import jax
import jax.numpy as jnp
from jax import lax
from jax.experimental import pallas as pl
from jax.experimental.pallas import tpu as pltpu
from jax.experimental.pallas import tpu_sc as plsc

B = 4096
F = 26
V = 100000
D = 64
H1 = 1024
H2 = 512

NC = 2
NS = 16
NW = NC * NS
TOT = B * F
RPW = TOT // NW
CH = 128
NCH = RPW // CH


def _sc_gather_body(vt, bt, fid, emb_out, bias_out, idx_v, row_v, bias_v,
                    sem, semb):
    wid = lax.axis_index("s") * NC + lax.axis_index("c")
    cbase = wid * NCH

    pltpu.sync_copy(fid.at[wid], idx_v)

    def bias_chunk(c, carry):
        pltpu.async_copy(bt.at[idx_v.at[c]], bias_v.at[c], semb).wait()
        return carry

    lax.fori_loop(0, NCH, bias_chunk, 0)
    pltpu.sync_copy(bias_v, bias_out.at[wid])

    def emb_chunk(c, carry):
        pltpu.async_copy(vt.at[idx_v.at[c]], row_v, sem).wait()
        pltpu.sync_copy(row_v, emb_out.at[pl.ds((cbase + c) * CH, CH)])
        return carry

    lax.fori_loop(0, NCH, emb_chunk, 0)


@functools.cache
def _sc_gather():
    return pl.kernel(
        _sc_gather_body,
        out_type=(
            jax.ShapeDtypeStruct((TOT, D), jnp.float32),
            jax.ShapeDtypeStruct((NW, NCH, CH), jnp.float32),
        ),
        mesh=plsc.VectorSubcoreMesh(core_axis_name="c", subcore_axis_name="s"),
        compiler_params=pltpu.CompilerParams(use_tc_tiling_on_sc=False),
        scratch_types=[
            pltpu.VMEM((NCH, CH), jnp.int32),
            pltpu.VMEM((CH, D), jnp.float32),
            pltpu.VMEM((NCH, CH), jnp.float32),
            pltpu.SemaphoreType.DMA,
            pltpu.SemaphoreType.DMA,
        ],
    )


def _mlp_body(x_ref, bs_ref, w1_ref, b1_ref, w2_ref, b2_ref, w3_ref, b3_ref,
              gb_ref, a_ref, z_ref):
    x = x_ref[...]
    s = jnp.dot(x, a_ref[...], preferred_element_type=jnp.float32)
    sumsq = jnp.sum(x * x, axis=1)
    order2 = jnp.sum(s * s, axis=1) - sumsq
    fm = 0.5 * order2 + jnp.sum(bs_ref[...], axis=1)

    h = jnp.dot(x, w1_ref[...], preferred_element_type=jnp.float32) + b1_ref[...]
    h = jnp.where(h > 0, h, 0.2 * h)
    h = jnp.dot(h, w2_ref[...], preferred_element_type=jnp.float32) + b2_ref[...]
    h = jnp.where(h > 0, h, 0.2 * h)
    z = jnp.dot(h, w3_ref[...], preferred_element_type=jnp.float32) + b3_ref[...]
    z_ref[...] = z + fm[:, None] + gb_ref[...]


BB = 256


def _mlp(x, bias2d, w1p, b1r, w2, b2r, w3, b3r, gbr, a, interpret=False):
    return pl.pallas_call(
        _mlp_body,
        grid=(B // BB,),
        in_specs=[
            pl.BlockSpec((BB, F * D), lambda i: (i, 0)),
            pl.BlockSpec((BB, F), lambda i: (i, 0)),
            pl.BlockSpec((F * D, H1), lambda i: (0, 0)),
            pl.BlockSpec((1, H1), lambda i: (0, 0)),
            pl.BlockSpec((H1, H2), lambda i: (0, 0)),
            pl.BlockSpec((1, H2), lambda i: (0, 0)),
            pl.BlockSpec((H2, 1), lambda i: (0, 0)),
            pl.BlockSpec((1, 1), lambda i: (0, 0)),
            pl.BlockSpec((1, 1), lambda i: (0, 0)),
            pl.BlockSpec((F * D, D), lambda i: (0, 0)),
        ],
        out_specs=pl.BlockSpec((BB, 1), lambda i: (i, 0)),
        out_shape=jax.ShapeDtypeStruct((B, 1), jnp.float32),
        interpret=interpret,
    )(x, bias2d, w1p, b1r, w2, b2r, w3, b3r, gbr, a)


def kernel(onehot_ids, v_tables, b_tables, W1, b1, W2, b2, W3, b3,
           global_bias):
    vt = v_tables.reshape(F * V, D)
    bt = b_tables.reshape(F * V)
    fid = (onehot_ids.astype(jnp.int32)
           + jnp.arange(F, dtype=jnp.int32)[None, :] * V)
    fid = fid.reshape(NW, NCH, CH)

    emb, bias = _sc_gather()(vt, bt, fid)

    x = emb.reshape(B, F * D)
    bias2d = bias.reshape(B, F)
    w1p = W1.reshape(D, F, H1).transpose(1, 0, 2).reshape(F * D, H1)
    a = jnp.tile(jnp.eye(D, dtype=jnp.float32), (F, 1))
    z = _mlp(x, bias2d, w1p, b1.reshape(1, H1), W2, b2.reshape(1, H2), W3,
             b3.reshape(1, 1), global_bias.reshape(1, 1), a)
    return z


# speedup vs baseline: 1.2396x; 1.0813x over previous
import functools

import jax
import jax.numpy as jnp
from jax import lax
from jax.experimental import pallas as pl
from jax.experimental.pallas import tpu as pltpu
from jax.experimental.pallas import tpu_sc as plsc

B = 4096
F = 26
V = 100000
D = 64
FD = F * D
H1 = 1024
H2 = 512

NC = 2
NS = 16
NW = NC * NS
CB = B // NW
CHR = 64
NBLK = FD // CHR


def _sc_gather_body(vt, bt, idx3, bidx3, xt_out, bias_out,
                    idx_v, got_v, bidx_v, bias_v, sem, semb):
    wid = lax.axis_index("s") * NC + lax.axis_index("c")

    pltpu.sync_copy(bidx3.at[wid], bidx_v)
    bh = [pltpu.async_copy(bt.at[bidx_v.at[c]], bias_v.at[c], semb)
          for c in range(F)]
    for h in bh:
        h.wait()
    pltpu.sync_copy(bias_v, bias_out.at[wid])

    def blk(c, carry):
        pltpu.sync_copy(idx3.at[wid, pl.ds(c * CHR, CHR)], idx_v)
        hs = [pltpu.async_copy(vt.at[idx_v.at[j]], got_v.at[j], sem)
              for j in range(CHR)]
        for h in hs:
            h.wait()
        pltpu.sync_copy(got_v, xt_out.at[wid, pl.ds(c * CHR, CHR)])
        return carry

    lax.fori_loop(0, NBLK, blk, 0)


@functools.cache
def _sc_gather():
    return pl.kernel(
        _sc_gather_body,
        out_type=(
            jax.ShapeDtypeStruct((NW, FD, CB), jnp.float32),
            jax.ShapeDtypeStruct((NW, F, CB), jnp.float32),
        ),
        mesh=plsc.VectorSubcoreMesh(core_axis_name="c", subcore_axis_name="s"),
        compiler_params=pltpu.CompilerParams(use_tc_tiling_on_sc=False),
        scratch_types=[
            pltpu.VMEM((CHR, CB), jnp.int32),
            pltpu.VMEM((CHR, CB), jnp.float32),
            pltpu.VMEM((F, CB), jnp.int32),
            pltpu.VMEM((F, CB), jnp.float32),
            pltpu.SemaphoreType.DMA,
            pltpu.SemaphoreType.DMA,
        ],
    )


def _mlp_body(x_ref, bs_ref, w1_ref, b1_ref, w2_ref, b2_ref, w3_ref, b3_ref,
              gb_ref, a_ref, z_ref):
    x = x_ref[0]
    s = jnp.dot(a_ref[...], x, preferred_element_type=jnp.float32)
    order2 = jnp.sum(s * s, axis=0) - jnp.sum(x * x, axis=0)
    fm = 0.5 * order2 + jnp.sum(bs_ref[0], axis=0)

    h = jnp.dot(w1_ref[...], x, preferred_element_type=jnp.float32) + b1_ref[...]
    h = jnp.where(h > 0, h, 0.2 * h)
    h = jnp.dot(w2_ref[...], h, preferred_element_type=jnp.float32) + b2_ref[...]
    h = jnp.where(h > 0, h, 0.2 * h)
    z = jnp.dot(w3_ref[...], h, preferred_element_type=jnp.float32) + b3_ref[...]
    z_ref[0] = z + fm[None, :] + gb_ref[...]


def _mlp(xt3, bias3, w1t, b1c, w2t, b2c, w3t, b3c, gbc, at, interpret=False):
    return pl.pallas_call(
        _mlp_body,
        grid=(NW,),
        in_specs=[
            pl.BlockSpec((1, FD, CB), lambda i: (i, 0, 0)),
            pl.BlockSpec((1, F, CB), lambda i: (i, 0, 0)),
            pl.BlockSpec((H1, FD), lambda i: (0, 0)),
            pl.BlockSpec((H1, 1), lambda i: (0, 0)),
            pl.BlockSpec((H2, H1), lambda i: (0, 0)),
            pl.BlockSpec((H2, 1), lambda i: (0, 0)),
            pl.BlockSpec((1, H2), lambda i: (0, 0)),
            pl.BlockSpec((1, 1), lambda i: (0, 0)),
            pl.BlockSpec((1, 1), lambda i: (0, 0)),
            pl.BlockSpec((D, FD), lambda i: (0, 0)),
        ],
        out_specs=pl.BlockSpec((1, 1, CB), lambda i: (i, 0, 0)),
        out_shape=jax.ShapeDtypeStruct((NW, 1, CB), jnp.float32),
        interpret=interpret,
    )(xt3, bias3, w1t, b1c, w2t, b2c, w3t, b3c, gbc, at)


def kernel(onehot_ids, v_tables, b_tables, W1, b1, W2, b2, W3, b3,
           global_bias):
    vt = v_tables.transpose(0, 2, 1).reshape(FD * V)
    bt = b_tables.reshape(F * V)

    ids_t = onehot_ids.astype(jnp.int32).T
    row_base = (jnp.arange(FD, dtype=jnp.int32) * V)[:, None]
    idx = row_base + jnp.repeat(ids_t, D, axis=0)
    idx3 = idx.reshape(FD, NW, CB).transpose(1, 0, 2)
    bidx = (jnp.arange(F, dtype=jnp.int32) * V)[:, None] + ids_t
    bidx3 = bidx.reshape(F, NW, CB).transpose(1, 0, 2)

    xt3, bias3 = _sc_gather()(vt, bt, idx3, bidx3)

    w1t = W1.reshape(D, F, H1).transpose(2, 1, 0).reshape(H1, FD)
    at = jnp.tile(jnp.eye(D, dtype=jnp.float32), (1, F))
    zt3 = _mlp(xt3, bias3, w1t, b1.reshape(H1, 1), W2.T, b2.reshape(H2, 1),
               W3.T, b3.reshape(1, 1), global_bias.reshape(1, 1), at)
    return zt3.reshape(B)[:, None]
